# baseline (speedup 1.0000x reference)
import jax
import jax.numpy as jnp
from jax.experimental import pallas as pl
from jax.experimental.pallas import tpu as pltpu

_BN = 512
_BK = 1024


def _body(x_ref, c_ref, idx_ref, dist_ref, x2_ref, c2_ref):
    i = pl.program_id(0)
    k = pl.program_id(1)
    nk = pl.num_programs(1)
    x = x_ref[...]
    c = c_ref[...]
    dot = jax.lax.dot_general(
        x, c, (((1,), (1,)), ((), ())),
        preferred_element_type=jnp.float32)

    @pl.when(k == 0)
    def _cache_x2():
        x2_ref[...] = jnp.sum(x * x, axis=1, keepdims=True)

    @pl.when(i == 0)
    def _cache_c2():
        c2_ref[k] = jnp.sum(c * c, axis=1)[None, :]

    x2 = x2_ref[...]
    c2 = c2_ref[k]
    d2 = (x2 + c2) - 2.0 * dot
    lmin = jnp.min(d2, axis=1, keepdims=True)
    iota = jax.lax.broadcasted_iota(jnp.int32, d2.shape, 1)
    lidx = k * _BK + jnp.min(
        jnp.where(d2 == lmin, iota, jnp.int32(2**31 - 1)),
        axis=1, keepdims=True)

    @pl.when(k == 0)
    def _init():
        dist_ref[...] = lmin
        idx_ref[...] = lidx

    @pl.when(k > 0)
    def _update():
        better = lmin < dist_ref[...]
        dist_ref[...] = jnp.where(better, lmin, dist_ref[...])
        idx_ref[...] = jnp.where(better, lidx, idx_ref[...])

    @pl.when(k == nk - 1)
    def _finalize():
        dist_ref[...] = jnp.sqrt(jnp.maximum(dist_ref[...], 1e-12))


def kernel(state, centroids):
    if state.ndim == 1:
        state = state[None, :]
    n, d = state.shape
    kk, _ = centroids.shape
    grid = (n // _BN, kk // _BK)
    idx2, dist2 = pl.pallas_call(
        _body,
        grid=grid,
        in_specs=[
            pl.BlockSpec((_BN, d), lambda i, j: (i, 0)),
            pl.BlockSpec((_BK, d), lambda i, j: (j, 0)),
        ],
        out_specs=[
            pl.BlockSpec((_BN, 1), lambda i, j: (i, 0)),
            pl.BlockSpec((_BN, 1), lambda i, j: (i, 0)),
        ],
        out_shape=[
            jax.ShapeDtypeStruct((n, 1), jnp.int32),
            jax.ShapeDtypeStruct((n, 1), jnp.float32),
        ],
        scratch_shapes=[
            pltpu.VMEM((_BN, 1), jnp.float32),
            pltpu.VMEM((kk // _BK, 1, _BK), jnp.float32),
        ],
    )(state, centroids)
    return idx2[:, 0], dist2[:, 0]


# speedup vs baseline: 1.2743x; 1.2743x over previous
import jax
import jax.numpy as jnp
from jax.experimental import pallas as pl

_BN = 512
_BK = 1024
_LANES = 128
_IMAX = 2**31 - 1


def _norms_body(x_ref, c_ref, x2_ref, c2_ref):
    x = x_ref[...]
    c = c_ref[...]
    x2_ref[...] = jnp.sum(x * x, axis=1, keepdims=True)
    c2_ref[...] = jnp.sum(c * c, axis=1)[None, :]


def _body(x_ref, c_ref, x2_ref, c2_ref, idx_ref, dist_ref):
    k = pl.program_id(1)
    nk = pl.num_programs(1)
    x = x_ref[...]
    c = c_ref[...]
    dot2 = jax.lax.dot_general(
        x + x, c, (((1,), (1,)), ((), ())),
        preferred_element_type=jnp.float32)
    x2 = x2_ref[...]
    c2 = c2_ref[...]

    run_val = (x2 + c2[:, :_LANES]) - dot2[:, :_LANES]
    run_cid = jnp.zeros(run_val.shape, jnp.int32)
    for ci in range(1, _BK // _LANES):
        sl = slice(ci * _LANES, (ci + 1) * _LANES)
        d2c = (x2 + c2[:, sl]) - dot2[:, sl]
        m = d2c < run_val
        run_val = jnp.where(m, d2c, run_val)
        run_cid = jnp.where(m, jnp.int32(ci), run_cid)

    gidx = (run_cid * _LANES + k * _BK
            + jax.lax.broadcasted_iota(jnp.int32, run_val.shape, 1))
    lmin = jnp.min(run_val, axis=1, keepdims=True)
    lidx = jnp.min(jnp.where(run_val == lmin, gidx, jnp.int32(_IMAX)),
                   axis=1, keepdims=True)

    @pl.when(k == 0)
    def _init():
        dist_ref[...] = lmin
        idx_ref[...] = lidx

    @pl.when(k > 0)
    def _update():
        better = lmin < dist_ref[...]
        dist_ref[...] = jnp.where(better, lmin, dist_ref[...])
        idx_ref[...] = jnp.where(better, lidx, idx_ref[...])

    @pl.when(k == nk - 1)
    def _finalize():
        dist_ref[...] = jnp.sqrt(jnp.maximum(dist_ref[...], 1e-12))


def kernel(state, centroids):
    if state.ndim == 1:
        state = state[None, :]
    n, d = state.shape
    kk, _ = centroids.shape

    x2, c2 = pl.pallas_call(
        _norms_body,
        out_shape=[
            jax.ShapeDtypeStruct((n, 1), jnp.float32),
            jax.ShapeDtypeStruct((1, kk), jnp.float32),
        ],
    )(state, centroids)

    grid = (n // _BN, kk // _BK)
    idx2, dist2 = pl.pallas_call(
        _body,
        grid=grid,
        in_specs=[
            pl.BlockSpec((_BN, d), lambda i, j: (i, 0)),
            pl.BlockSpec((_BK, d), lambda i, j: (j, 0)),
            pl.BlockSpec((_BN, 1), lambda i, j: (i, 0)),
            pl.BlockSpec((1, _BK), lambda i, j: (0, j)),
        ],
        out_specs=[
            pl.BlockSpec((_BN, 1), lambda i, j: (i, 0)),
            pl.BlockSpec((_BN, 1), lambda i, j: (i, 0)),
        ],
        out_shape=[
            jax.ShapeDtypeStruct((n, 1), jnp.int32),
            jax.ShapeDtypeStruct((n, 1), jnp.float32),
        ],
    )(state, centroids, x2, c2)
    return idx2[:, 0], dist2[:, 0]


# speedup vs baseline: 2.3795x; 1.8673x over previous
import jax
import jax.numpy as jnp
from jax.experimental import pallas as pl

_BN = 512
_LANES = 128
_IMAX = 2**31 - 1


def _norms_body(x_ref, c_ref, x2_ref, c2_ref):
    x = x_ref[...]
    c = c_ref[...]
    x2_ref[...] = jnp.sum(x * x, axis=1, keepdims=True)
    c2_ref[...] = jnp.sum(c * c, axis=1)[None, :]


def _body(x_ref, c_ref, x2_ref, c2_ref, idx_ref, dist_ref):
    x = x_ref[...]
    c = c_ref[...]
    dot2 = jax.lax.dot_general(
        x + x, c, (((1,), (1,)), ((), ())),
        preferred_element_type=jnp.float32)
    x2 = x2_ref[...]
    c2 = c2_ref[...]

    kk = c.shape[0]
    run_val = (x2 + c2[:, :_LANES]) - dot2[:, :_LANES]
    run_cid = jnp.zeros(run_val.shape, jnp.int32)
    for ci in range(1, kk // _LANES):
        sl = slice(ci * _LANES, (ci + 1) * _LANES)
        d2c = (x2 + c2[:, sl]) - dot2[:, sl]
        m = d2c < run_val
        run_val = jnp.where(m, d2c, run_val)
        run_cid = jnp.where(m, jnp.int32(ci), run_cid)

    gidx = (run_cid * _LANES
            + jax.lax.broadcasted_iota(jnp.int32, run_val.shape, 1))
    lmin = jnp.min(run_val, axis=1, keepdims=True)
    idx_ref[...] = jnp.min(
        jnp.where(run_val == lmin, gidx, jnp.int32(_IMAX)),
        axis=1, keepdims=True)
    dist_ref[...] = jnp.sqrt(jnp.maximum(lmin, 1e-12))


def kernel(state, centroids):
    if state.ndim == 1:
        state = state[None, :]
    n, d = state.shape
    kk, _ = centroids.shape

    x2, c2 = pl.pallas_call(
        _norms_body,
        out_shape=[
            jax.ShapeDtypeStruct((n, 1), jnp.float32),
            jax.ShapeDtypeStruct((1, kk), jnp.float32),
        ],
    )(state, centroids)

    grid = (n // _BN,)
    idx2, dist2 = pl.pallas_call(
        _body,
        grid=grid,
        in_specs=[
            pl.BlockSpec((_BN, d), lambda i: (i, 0)),
            pl.BlockSpec((kk, d), lambda i: (0, 0)),
            pl.BlockSpec((_BN, 1), lambda i: (i, 0)),
            pl.BlockSpec((1, kk), lambda i: (0, 0)),
        ],
        out_specs=[
            pl.BlockSpec((_BN, 1), lambda i: (i, 0)),
            pl.BlockSpec((_BN, 1), lambda i: (i, 0)),
        ],
        out_shape=[
            jax.ShapeDtypeStruct((n, 1), jnp.int32),
            jax.ShapeDtypeStruct((n, 1), jnp.float32),
        ],
    )(state, centroids, x2, c2)
    return idx2[:, 0], dist2[:, 0]


# speedup vs baseline: 2.3900x; 1.0044x over previous
import jax
import jax.numpy as jnp
from jax.experimental import pallas as pl

_BN = 512
_SUBROWS = 128
_LANES = 128
_IMAX = 2**31 - 1


def _norms_body(x_ref, c_ref, x2_ref, c2_ref):
    x = x_ref[...]
    c = c_ref[...]
    x2_ref[...] = jnp.sum(x * x, axis=1, keepdims=True)
    c2_ref[...] = jnp.sum(c * c, axis=1)[None, :]


def _body(x_ref, c_ref, x2_ref, c2_ref, idx_ref, dist_ref):
    x = x_ref[...]
    c = c_ref[...]
    dot2 = jax.lax.dot_general(
        x + x, c, (((1,), (1,)), ((), ())),
        preferred_element_type=jnp.float32)
    x2 = x2_ref[...]
    c2 = c2_ref[...]

    kk = c.shape[0]
    bn = x.shape[0]
    for rb in range(bn // _SUBROWS):
        rs = slice(rb * _SUBROWS, (rb + 1) * _SUBROWS)
        x2b = x2[rs, :]
        run_val = (x2b + c2[:, :_LANES]) - dot2[rs, :_LANES]
        run_cid = jnp.zeros(run_val.shape, jnp.int32)
        for ci in range(1, kk // _LANES):
            sl = slice(ci * _LANES, (ci + 1) * _LANES)
            d2c = (x2b + c2[:, sl]) - dot2[rs, sl]
            m = d2c < run_val
            run_val = jnp.where(m, d2c, run_val)
            run_cid = jnp.where(m, jnp.int32(ci), run_cid)

        gidx = (run_cid * _LANES
                + jax.lax.broadcasted_iota(jnp.int32, run_val.shape, 1))
        lmin = jnp.min(run_val, axis=1, keepdims=True)
        idx_ref[rs, :] = jnp.min(
            jnp.where(run_val == lmin, gidx, jnp.int32(_IMAX)),
            axis=1, keepdims=True)
        dist_ref[rs, :] = jnp.sqrt(jnp.maximum(lmin, 1e-12))


def kernel(state, centroids):
    if state.ndim == 1:
        state = state[None, :]
    n, d = state.shape
    kk, _ = centroids.shape

    x2, c2 = pl.pallas_call(
        _norms_body,
        out_shape=[
            jax.ShapeDtypeStruct((n, 1), jnp.float32),
            jax.ShapeDtypeStruct((1, kk), jnp.float32),
        ],
    )(state, centroids)

    grid = (n // _BN,)
    idx2, dist2 = pl.pallas_call(
        _body,
        grid=grid,
        in_specs=[
            pl.BlockSpec((_BN, d), lambda i: (i, 0)),
            pl.BlockSpec((kk, d), lambda i: (0, 0)),
            pl.BlockSpec((_BN, 1), lambda i: (i, 0)),
            pl.BlockSpec((1, kk), lambda i: (0, 0)),
        ],
        out_specs=[
            pl.BlockSpec((_BN, 1), lambda i: (i, 0)),
            pl.BlockSpec((_BN, 1), lambda i: (i, 0)),
        ],
        out_shape=[
            jax.ShapeDtypeStruct((n, 1), jnp.int32),
            jax.ShapeDtypeStruct((n, 1), jnp.float32),
        ],
    )(state, centroids, x2, c2)
    return idx2[:, 0], dist2[:, 0]
